# baseline (speedup 1.0000x reference)
import functools

import jax
import jax.numpy as jnp
from jax import lax
from jax.experimental import pallas as pl
from jax.experimental.pallas import tpu as pltpu
from jax.experimental.pallas import tpu_sc as plsc

_D = 1024
_B = 4 * 4096
_SCALE = 32.0
_NC = 2
_NS = 16
_NW = _NC * _NS
_BPW = _B // _NW
_CHUNK = 32
_NCHUNK = _BPW // _CHUNK
_LANES = 16
_NTRIPLE = 5


def _emb_body(x_hbm, table_hbm, out_hbm, idx_v,
              buf0, buf1, buf2, gsem0, gsem1, gsem2, osem0, osem1, osem2):
    wid = lax.axis_index("s") * _NC + lax.axis_index("c")
    base = wid * _BPW
    wpr = 4096 // _BPW
    pltpu.sync_copy(
        x_hbm.at[wid // wpr, pl.ds((wid % wpr) * _BPW, _BPW)], idx_v)

    bufs = (buf0, buf1, buf2)
    gsems = (gsem0, gsem1, gsem2)
    osems = (osem0, osem1, osem2)

    def gstart(c, b):
        pltpu.async_copy(
            table_hbm.at[idx_v.at[pl.ds(c * _CHUNK, _CHUNK)]],
            bufs[b],
            gsems[b],
        )

    def gwait(b):
        pltpu.make_async_copy(
            table_hbm.at[idx_v.at[pl.ds(0, _CHUNK)]],
            bufs[b],
            gsems[b],
        ).wait()

    def ostart(c, b):
        pltpu.async_copy(
            bufs[b],
            out_hbm.at[pl.ds(base + c * _CHUNK, _CHUNK)],
            osems[b],
        )

    def owait(b):
        pltpu.make_async_copy(
            bufs[b],
            out_hbm.at[pl.ds(0, _CHUNK)],
            osems[b],
        ).wait()

    def scale(b):
        buf = bufs[b]

        def row_body(r, carry):
            @plsc.parallel_loop(0, _D // _LANES, step=1, unroll=8)
            def _(j):
                sl = pl.ds(j * _LANES, _LANES)
                buf[r, sl] = buf[r, sl] * _SCALE

            return carry

        lax.fori_loop(0, _CHUNK, row_body, 0)

    gstart(0, 0)
    gstart(1, 1)

    def triple(i, carry):
        c = i * 3
        gwait(0)
        scale(0)
        ostart(c, 0)

        @pl.when(i >= 1)
        def _():
            owait(2)

        gstart(c + 2, 2)
        gwait(1)
        scale(1)
        ostart(c + 1, 1)
        owait(0)
        gstart(c + 3, 0)
        gwait(2)
        scale(2)
        ostart(c + 2, 2)

        @pl.when(i <= _NTRIPLE - 2)
        def _():
            owait(1)
            gstart(c + 4, 1)

        return carry

    lax.fori_loop(0, _NTRIPLE, triple, 0)

    gwait(0)
    scale(0)
    ostart(_NCHUNK - 1, 0)
    owait(1)
    owait(2)
    owait(0)


@jax.jit
def _emb(x2d, table):
    mesh = plsc.VectorSubcoreMesh(core_axis_name="c", subcore_axis_name="s")
    run = functools.partial(
        pl.kernel,
        mesh=mesh,
        out_type=jax.ShapeDtypeStruct((_B, _D), jnp.float32),
        scratch_types=[
            pltpu.VMEM((_BPW,), jnp.int32),
            pltpu.VMEM((_CHUNK, _D), jnp.float32),
            pltpu.VMEM((_CHUNK, _D), jnp.float32),
            pltpu.VMEM((_CHUNK, _D), jnp.float32),
            pltpu.SemaphoreType.DMA,
            pltpu.SemaphoreType.DMA,
            pltpu.SemaphoreType.DMA,
            pltpu.SemaphoreType.DMA,
            pltpu.SemaphoreType.DMA,
            pltpu.SemaphoreType.DMA,
        ],
    )(_emb_body)
    return run(x2d, table)


def kernel(x, table):
    out = _emb(x.astype(jnp.int32), table)
    return out.reshape(x.shape + (_D,))


# speedup vs baseline: 1.0015x; 1.0015x over previous
import functools

import jax
import jax.numpy as jnp
from jax import lax
from jax.experimental import pallas as pl
from jax.experimental.pallas import tpu as pltpu
from jax.experimental.pallas import tpu_sc as plsc

_D = 1024
_B = 4 * 4096
_SCALE = 32.0
_NC = 2
_NS = 16
_NW = _NC * _NS
_BPW = _B // _NW
_CHUNK = 16
_NCHUNK = _BPW // _CHUNK
_NBUF = 4
_NGROUP = _NCHUNK // _NBUF
_LANES = 16


def _emb_body(x_hbm, table_hbm, out_hbm, idx_v,
              buf0, buf1, buf2, buf3,
              gsem0, gsem1, gsem2, gsem3, osem0, osem1, osem2, osem3):
    wid = lax.axis_index("s") * _NC + lax.axis_index("c")
    base = wid * _BPW
    wpr = 4096 // _BPW
    pltpu.sync_copy(
        x_hbm.at[wid // wpr, pl.ds((wid % wpr) * _BPW, _BPW)], idx_v)

    bufs = (buf0, buf1, buf2, buf3)
    gsems = (gsem0, gsem1, gsem2, gsem3)
    osems = (osem0, osem1, osem2, osem3)

    def gstart(c, b):
        pltpu.async_copy(
            table_hbm.at[idx_v.at[pl.ds(c * _CHUNK, _CHUNK)]],
            bufs[b],
            gsems[b],
        )

    def gwait(b):
        pltpu.make_async_copy(
            table_hbm.at[idx_v.at[pl.ds(0, _CHUNK)]],
            bufs[b],
            gsems[b],
        ).wait()

    def ostart(c, b):
        pltpu.async_copy(
            bufs[b],
            out_hbm.at[pl.ds(base + c * _CHUNK, _CHUNK)],
            osems[b],
        )

    def owait(b):
        pltpu.make_async_copy(
            bufs[b],
            out_hbm.at[pl.ds(0, _CHUNK)],
            osems[b],
        ).wait()

    def scale(b):
        buf = bufs[b]

        def row_body(r, carry):
            @plsc.parallel_loop(0, _D // _LANES, step=1, unroll=8)
            def _(j):
                sl = pl.ds(j * _LANES, _LANES)
                buf[r, sl] = buf[r, sl] * _SCALE

            return carry

        lax.fori_loop(0, _CHUNK, row_body, 0)

    gstart(0, 0)
    gstart(1, 1)

    def group(i, carry):
        c = i * _NBUF
        for k in range(_NBUF):
            b = k
            nb = (k + 2) % _NBUF
            gwait(b)
            scale(b)
            ostart(c + k, b)
            if k < 2:
                @pl.when(i >= 1)
                def _():
                    owait(nb)

                gstart(c + k + 2, nb)
            else:
                @pl.when(i <= _NGROUP - 2)
                def _():
                    owait(nb)
                    gstart(c + k + 2, nb)

        return carry

    lax.fori_loop(0, _NGROUP, group, 0)

    for b in range(_NBUF):
        owait(b)


@jax.jit
def _emb(x2d, table):
    mesh = plsc.VectorSubcoreMesh(core_axis_name="c", subcore_axis_name="s")
    run = functools.partial(
        pl.kernel,
        mesh=mesh,
        out_type=jax.ShapeDtypeStruct((_B, _D), jnp.float32),
        scratch_types=[
            pltpu.VMEM((_BPW,), jnp.int32),
            pltpu.VMEM((_CHUNK, _D), jnp.float32),
            pltpu.VMEM((_CHUNK, _D), jnp.float32),
            pltpu.VMEM((_CHUNK, _D), jnp.float32),
            pltpu.VMEM((_CHUNK, _D), jnp.float32),
            pltpu.SemaphoreType.DMA,
            pltpu.SemaphoreType.DMA,
            pltpu.SemaphoreType.DMA,
            pltpu.SemaphoreType.DMA,
            pltpu.SemaphoreType.DMA,
            pltpu.SemaphoreType.DMA,
            pltpu.SemaphoreType.DMA,
            pltpu.SemaphoreType.DMA,
        ],
    )(_emb_body)
    return run(x2d, table)


def kernel(x, table):
    out = _emb(x.astype(jnp.int32), table)
    return out.reshape(x.shape + (_D,))


# speedup vs baseline: 1.0223x; 1.0208x over previous
import functools

import jax
import jax.numpy as jnp
from jax import lax
from jax.experimental import pallas as pl
from jax.experimental.pallas import tpu as pltpu
from jax.experimental.pallas import tpu_sc as plsc

_D = 1024
_B = 4 * 4096
_SCALE = 32.0
_NC = 2
_NS = 16
_NW = _NC * _NS
_BPW = _B // _NW
_CHUNK = 16
_NCHUNK = _BPW // _CHUNK
_NBUF = 4
_NGROUP = _NCHUNK // _NBUF
_LANES = 16


_NSLOT = 2


def _emb_body(x_hbm, table_hbm, out_hbm, idx_v,
              buf0, buf1, buf2, buf3, sp,
              gsem0, gsem1, gsem2, gsem3,
              ssem0, ssem1, ssem2, ssem3,
              osem0, osem1):
    cid = lax.axis_index("c")
    sid = lax.axis_index("s")
    wid = sid * _NC + cid
    base = wid * _BPW
    wpr = 4096 // _BPW
    pltpu.sync_copy(
        x_hbm.at[wid // wpr, pl.ds((wid % wpr) * _BPW, _BPW)], idx_v)

    bufs = (buf0, buf1, buf2, buf3)
    gsems = (gsem0, gsem1, gsem2, gsem3)
    ssems = (ssem0, ssem1, ssem2, ssem3)
    osems = (osem0, osem1)

    def gstart(c, b):
        pltpu.async_copy(
            table_hbm.at[idx_v.at[pl.ds(c * _CHUNK, _CHUNK)]],
            bufs[b],
            gsems[b],
        )

    def gwait(b):
        pltpu.make_async_copy(
            table_hbm.at[idx_v.at[pl.ds(0, _CHUNK)]],
            bufs[b],
            gsems[b],
        ).wait()

    def sstart(b, s):
        pltpu.async_copy(bufs[b], sp.at[sid, s], ssems[b])

    def swait(b, s):
        pltpu.make_async_copy(bufs[b], sp.at[sid, s], ssems[b]).wait()

    def ostart(c, s):
        pltpu.async_copy(
            sp.at[sid, s],
            out_hbm.at[pl.ds(base + c * _CHUNK, _CHUNK)],
            osems[s],
        )

    def owait(s):
        pltpu.make_async_copy(
            sp.at[sid, s],
            out_hbm.at[pl.ds(0, _CHUNK)],
            osems[s],
        ).wait()

    def scale(b):
        buf = bufs[b]

        def row_body(r, carry):
            @plsc.parallel_loop(0, _D // _LANES, step=1, unroll=8)
            def _(j):
                sl = pl.ds(j * _LANES, _LANES)
                buf[r, sl] = buf[r, sl] * _SCALE

            return carry

        lax.fori_loop(0, _CHUNK, row_body, 0)

    gstart(0, 0)
    gstart(1, 1)

    def group(i, carry):
        c0 = i * _NBUF
        for k in range(_NBUF):
            c = c0 + k
            b = k
            s = k % _NSLOT
            gwait(b)
            scale(b)

            if k < 2:
                @pl.when(i >= 1)
                def _():
                    owait(s)
            else:
                owait(s)

            sstart(b, s)

            pb = (k - 1) % _NBUF
            ps = (k - 1) % _NSLOT
            if k == 0:
                @pl.when(i >= 1)
                def _():
                    swait(pb, ps)
                    ostart(c - 1, ps)
            else:
                swait(pb, ps)
                ostart(c - 1, ps)

            nb = (k + 2) % _NBUF
            if k < 2:
                gstart(c + 2, nb)
            else:
                @pl.when(i <= _NGROUP - 2)
                def _():
                    gstart(c + 2, nb)

        return carry

    lax.fori_loop(0, _NGROUP, group, 0)

    swait(3, 1)
    ostart(_NCHUNK - 1, 1)
    owait(0)
    owait(1)


@jax.jit
def _emb(x2d, table):
    mesh = plsc.VectorSubcoreMesh(core_axis_name="c", subcore_axis_name="s")
    run = functools.partial(
        pl.kernel,
        mesh=mesh,
        out_type=jax.ShapeDtypeStruct((_B, _D), jnp.float32),
        scratch_types=[
            pltpu.VMEM((_BPW,), jnp.int32),
            pltpu.VMEM((_CHUNK, _D), jnp.float32),
            pltpu.VMEM((_CHUNK, _D), jnp.float32),
            pltpu.VMEM((_CHUNK, _D), jnp.float32),
            pltpu.VMEM((_CHUNK, _D), jnp.float32),
            pltpu.VMEM_SHARED((_NS, _NSLOT, _CHUNK, _D), jnp.float32),
            pltpu.SemaphoreType.DMA,
            pltpu.SemaphoreType.DMA,
            pltpu.SemaphoreType.DMA,
            pltpu.SemaphoreType.DMA,
            pltpu.SemaphoreType.DMA,
            pltpu.SemaphoreType.DMA,
            pltpu.SemaphoreType.DMA,
            pltpu.SemaphoreType.DMA,
            pltpu.SemaphoreType.DMA,
            pltpu.SemaphoreType.DMA,
        ],
    )(_emb_body)
    return run(x2d, table)


def kernel(x, table):
    out = _emb(x.astype(jnp.int32), table)
    return out.reshape(x.shape + (_D,))
